# baseline (speedup 1.0000x reference)
import functools

import jax
import jax.numpy as jnp
from jax import lax
from jax.experimental import pallas as pl
from jax.experimental.pallas import tpu as pltpu
from jax.experimental.pallas import tpu_sc as plsc

_NC, _NS = 2, 16
_NW = _NC * _NS
_SPLITS = ((0, 104), (104, 96))
_LANES = 16
_NBUF = 3
_LEAD = 2


@functools.lru_cache(maxsize=None)
def _make_kernel(n_rows, seq, hidden):
    k_per_w = n_rows // _NW
    n_tail = _LEAD + (k_per_w - 2 * _LEAD) % _NBUF
    n_steady = k_per_w - _LEAD - n_tail
    assert n_steady % _NBUF == 0
    assert _SPLITS[-1][0] + _SPLITS[-1][1] == seq
    mesh = plsc.VectorSubcoreMesh(
        core_axis_name="c", subcore_axis_name="s",
        num_cores=_NC, num_subcores=_NS)

    @functools.partial(
        pl.kernel,
        out_type=jax.ShapeDtypeStruct((n_rows, seq, 2 * hidden), jnp.float32),
        mesh=mesh,
        compiler_params=pltpu.CompilerParams(use_tc_tiling_on_sc=False),
        scratch_types=[
            pltpu.VMEM((k_per_w * seq,), jnp.int32),
            pltpu.VMEM((seq * hidden,), jnp.float32),
            pltpu.VMEM((_NBUF, seq, 2 * hidden), jnp.float32),
        ] + [pltpu.SemaphoreType.DMA] * (2 * _NBUF),
    )
    def k(idx_hbm, tok_hbm, pos_hbm, out_hbm, idx_v, pos_v, rows_v, *sems):
        gsems, ssems = sems[:_NBUF], sems[_NBUF:]
        wid = lax.axis_index("s") * _NC + lax.axis_index("c")
        base = wid * k_per_w
        pltpu.sync_copy(idx_hbm.at[pl.ds(base * seq, k_per_w * seq)], idx_v)
        pltpu.sync_copy(pos_hbm, pos_v)

        def gathers(kchunk, slot):
            return [
                pltpu.make_async_copy(
                    tok_hbm.at[idx_v.at[pl.ds(kchunk * seq + off, width)]],
                    rows_v.at[slot, pl.ds(off, width)],
                    gsems[slot])
                for off, width in _SPLITS]

        def scatter(kchunk, slot):
            return pltpu.make_async_copy(
                rows_v.at[slot, slice(None), pl.ds(0, hidden)],
                out_hbm.at[base + kchunk, slice(None), pl.ds(0, hidden)],
                ssems[slot])

        def add_pos(slot):
            def add_row(i, _):
                for j in range(hidden // _LANES):
                    plsc.addupdate(
                        rows_v.at[slot, i, pl.ds(j * _LANES, _LANES)],
                        pos_v[pl.ds(i * hidden + j * _LANES, _LANES)])
                return 0
            lax.fori_loop(0, seq, add_row, 0, unroll=4)

        def visit(kchunk, b, tail=False):
            for c in gathers(kchunk, b):
                c.wait()
            add_pos(b)
            scatter(kchunk, b).start()
            scatter(kchunk, b).wait()
            if not tail:
                for c in gathers(kchunk + _LEAD, (b + _LEAD) % _NBUF):
                    c.start()

        for p in range(_LEAD):
            for c in gathers(p, p):
                c.start()
        for p in range(_LEAD):
            visit(p, p % _NBUF)

        def steady(kk, _):
            k0 = _LEAD + kk * _NBUF
            for off in range(_NBUF):
                visit(k0 + off, (_LEAD + off) % _NBUF)
            return 0
        lax.fori_loop(0, n_steady // _NBUF, steady, 0)

        for p in range(k_per_w - n_tail, k_per_w):
            visit(p, p % _NBUF, tail=p + _LEAD >= k_per_w)

    return k


def kernel(batch_input_idx, token_table, position_table):
    b, s = batch_input_idx.shape
    v, hidden = token_table.shape
    idx = batch_input_idx.astype(jnp.int32).reshape(-1)
    pos = position_table[:s].reshape(-1)
    tok = jnp.pad(token_table, ((0, 0), (0, hidden)))
    out = _make_kernel(b, s, hidden)(idx, tok, pos)
    return out[:, :, :hidden]


# speedup vs baseline: 1.2514x; 1.2514x over previous
import functools

import jax
import jax.numpy as jnp
from jax import lax
from jax.experimental import pallas as pl
from jax.experimental.pallas import tpu as pltpu
from jax.experimental.pallas import tpu_sc as plsc

_NC, _NS = 2, 16
_NW = _NC * _NS
_SPLITS = ((0, 104), (104, 96))
_LANES = 16
_NBUF = 4
_LEAD = 2


@functools.lru_cache(maxsize=None)
def _make_kernel(n_rows, seq, hidden):
    k_per_w = n_rows // _NW
    n_tail = _LEAD + (k_per_w - 2 * _LEAD) % _NBUF
    n_steady = k_per_w - _LEAD - n_tail
    assert n_steady % _NBUF == 0
    assert _SPLITS[-1][0] + _SPLITS[-1][1] == seq
    mesh = plsc.VectorSubcoreMesh(
        core_axis_name="c", subcore_axis_name="s",
        num_cores=_NC, num_subcores=_NS)

    @functools.partial(
        pl.kernel,
        out_type=jax.ShapeDtypeStruct((n_rows, seq, 2 * hidden), jnp.float32),
        mesh=mesh,
        compiler_params=pltpu.CompilerParams(use_tc_tiling_on_sc=False),
        scratch_types=[
            pltpu.VMEM((k_per_w * seq,), jnp.int32),
            pltpu.VMEM((seq * hidden,), jnp.float32),
            pltpu.VMEM((_NBUF, seq, hidden), jnp.float32),
        ] + [pltpu.SemaphoreType.DMA] * (2 * _NBUF),
    )
    def k(idx_hbm, tok_hbm, pos_hbm, out_hbm, idx_v, pos_v, rows_v, *sems):
        gsems, ssems = sems[:_NBUF], sems[_NBUF:]
        wid = lax.axis_index("s") * _NC + lax.axis_index("c")
        base = wid * k_per_w
        pltpu.sync_copy(idx_hbm.at[pl.ds(base * seq, k_per_w * seq)], idx_v)
        pltpu.sync_copy(pos_hbm, pos_v)

        def gathers(kchunk, slot):
            return [
                pltpu.make_async_copy(
                    tok_hbm.at[idx_v.at[pl.ds(kchunk * seq + off, width)]],
                    rows_v.at[slot, pl.ds(off, width)],
                    gsems[slot])
                for off, width in _SPLITS]

        def scatter(kchunk, slot):
            return pltpu.make_async_copy(
                rows_v.at[slot],
                out_hbm.at[base + kchunk, slice(None), pl.ds(0, hidden)],
                ssems[slot])

        def add_pos(slot):
            def add_row(i, _):
                for j in range(hidden // _LANES):
                    plsc.addupdate(
                        rows_v.at[slot, i, pl.ds(j * _LANES, _LANES)],
                        pos_v[pl.ds(i * hidden + j * _LANES, _LANES)])
                return 0
            lax.fori_loop(0, seq, add_row, 0, unroll=4)

        def visit(kchunk, b, tail=False):
            for c in gathers(kchunk, b):
                c.wait()
            add_pos(b)
            scatter(kchunk, b).start()
            scatter(kchunk, b).wait()
            if not tail:
                for c in gathers(kchunk + _LEAD, (b + _LEAD) % _NBUF):
                    c.start()

        for p in range(_LEAD):
            for c in gathers(p, p):
                c.start()
        for p in range(_LEAD):
            visit(p, p % _NBUF)

        def steady(kk, _):
            k0 = _LEAD + kk * _NBUF
            for off in range(_NBUF):
                visit(k0 + off, (_LEAD + off) % _NBUF)
            return 0
        lax.fori_loop(0, n_steady // _NBUF, steady, 0)

        for p in range(k_per_w - n_tail, k_per_w):
            visit(p, p % _NBUF, tail=p + _LEAD >= k_per_w)

    return k


def kernel(batch_input_idx, token_table, position_table):
    b, s = batch_input_idx.shape
    v, hidden = token_table.shape
    idx = batch_input_idx.astype(jnp.int32).reshape(-1)
    pos = position_table[:s].reshape(-1)
    tok = lax.optimization_barrier(token_table.reshape(-1)).reshape(v, hidden)
    out = _make_kernel(b, s, hidden)(idx, tok, pos)
    return out[:, :, :hidden]


# speedup vs baseline: 1.2863x; 1.0279x over previous
import functools

import jax
import jax.numpy as jnp
from jax import lax
from jax.experimental import pallas as pl
from jax.experimental.pallas import tpu as pltpu
from jax.experimental.pallas import tpu_sc as plsc

_NC, _NS = 2, 16
_NW = _NC * _NS
_SPLITS = ((0, 104), (104, 96))
_LANES = 16
_NBUF = 4
_LEAD = 2


@functools.lru_cache(maxsize=None)
def _make_kernel(n_rows, seq, hidden):
    k_per_w = n_rows // _NW
    n_tail = _LEAD + (k_per_w - 2 * _LEAD) % _NBUF
    n_steady = k_per_w - _LEAD - n_tail
    assert n_steady % _NBUF == 0
    assert _SPLITS[-1][0] + _SPLITS[-1][1] == seq
    mesh = plsc.VectorSubcoreMesh(
        core_axis_name="c", subcore_axis_name="s",
        num_cores=_NC, num_subcores=_NS)

    @functools.partial(
        pl.kernel,
        out_type=jax.ShapeDtypeStruct((n_rows, seq, 2 * hidden), jnp.float32),
        mesh=mesh,
        compiler_params=pltpu.CompilerParams(use_tc_tiling_on_sc=False),
        scratch_types=[
            pltpu.VMEM((k_per_w * seq,), jnp.int32),
            pltpu.VMEM((seq * hidden,), jnp.float32),
            pltpu.VMEM((_NBUF, seq, hidden), jnp.float32),
        ] + [pltpu.SemaphoreType.DMA] * (2 * _NBUF),
    )
    def k(idx_hbm, tok_hbm, pos_hbm, out_hbm, idx_v, pos_v, rows_v, *sems):
        gsems, ssems = sems[:_NBUF], sems[_NBUF:]
        wid = lax.axis_index("s") * _NC + lax.axis_index("c")
        base = wid * k_per_w
        pltpu.sync_copy(idx_hbm.at[pl.ds(base * seq, k_per_w * seq)], idx_v)
        pltpu.sync_copy(pos_hbm, pos_v)

        def gathers(kchunk, slot):
            return [
                pltpu.make_async_copy(
                    tok_hbm.at[idx_v.at[pl.ds(kchunk * seq + off, width)]],
                    rows_v.at[slot, pl.ds(off, width)],
                    gsems[slot])
                for off, width in _SPLITS]

        def scatter(kchunk, slot):
            return pltpu.make_async_copy(
                rows_v.at[slot],
                out_hbm.at[base + kchunk, slice(None), pl.ds(0, hidden)],
                ssems[slot])

        def add_pos(slot):
            def add_row(i, _):
                for j in range(hidden // _LANES):
                    plsc.addupdate(
                        rows_v.at[slot, i, pl.ds(j * _LANES, _LANES)],
                        pos_v[pl.ds(i * hidden + j * _LANES, _LANES)])
                return 0
            lax.fori_loop(0, seq, add_row, 0, unroll=4)

        def visit(kchunk, b, tail=False, first=False):
            for c in gathers(kchunk, b):
                c.wait()
            add_pos(b)
            scatter(kchunk, b).start()
            if not first:
                scatter(kchunk - 1, (b - 1) % _NBUF).wait()
            if not tail:
                for c in gathers(kchunk + _LEAD, (b + _LEAD) % _NBUF):
                    c.start()

        for p in range(_LEAD):
            for c in gathers(p, p):
                c.start()
        for p in range(_LEAD):
            visit(p, p % _NBUF, first=p == 0)

        def steady(kk, _):
            k0 = _LEAD + kk * _NBUF
            for off in range(_NBUF):
                visit(k0 + off, (_LEAD + off) % _NBUF)
            return 0
        lax.fori_loop(0, n_steady // _NBUF, steady, 0)

        for p in range(k_per_w - n_tail, k_per_w):
            visit(p, p % _NBUF, tail=p + _LEAD >= k_per_w)
        scatter(k_per_w - 1, (k_per_w - 1) % _NBUF).wait()

    return k


def kernel(batch_input_idx, token_table, position_table):
    b, s = batch_input_idx.shape
    v, hidden = token_table.shape
    idx = batch_input_idx.astype(jnp.int32).reshape(-1)
    pos = position_table[:s].reshape(-1)
    tok = lax.optimization_barrier(token_table.reshape(-1)).reshape(v, hidden)
    out = _make_kernel(b, s, hidden)(idx, tok, pos)
    return out[:, :, :hidden]


# speedup vs baseline: 1.3038x; 1.0136x over previous
import functools

import jax
import jax.numpy as jnp
from jax import lax
from jax.experimental import pallas as pl
from jax.experimental.pallas import tpu as pltpu
from jax.experimental.pallas import tpu_sc as plsc

_NC, _NS = 2, 16
_NW = _NC * _NS
_SPLITS = ((0, 104), (104, 96))
_LANES = 16
_NBUF = 6
_LEAD = 3
_SLAG = 2


@functools.lru_cache(maxsize=None)
def _make_kernel(n_rows, seq, hidden):
    k_per_w = n_rows // _NW
    n_tail = _LEAD + (k_per_w - 2 * _LEAD) % _NBUF
    n_steady = k_per_w - _LEAD - n_tail
    assert n_steady % _NBUF == 0
    assert _SPLITS[-1][0] + _SPLITS[-1][1] == seq
    mesh = plsc.VectorSubcoreMesh(
        core_axis_name="c", subcore_axis_name="s",
        num_cores=_NC, num_subcores=_NS)

    @functools.partial(
        pl.kernel,
        out_type=jax.ShapeDtypeStruct((n_rows, seq, 2 * hidden), jnp.float32),
        mesh=mesh,
        compiler_params=pltpu.CompilerParams(use_tc_tiling_on_sc=False),
        scratch_types=[
            pltpu.VMEM((k_per_w * seq,), jnp.int32),
            pltpu.VMEM((seq * hidden,), jnp.float32),
            pltpu.VMEM((_NBUF, seq, hidden), jnp.float32),
        ] + [pltpu.SemaphoreType.DMA] * (2 * _NBUF),
    )
    def k(idx_hbm, tok_hbm, pos_hbm, out_hbm, idx_v, pos_v, rows_v, *sems):
        gsems, ssems = sems[:_NBUF], sems[_NBUF:]
        wid = lax.axis_index("s") * _NC + lax.axis_index("c")
        base = wid * k_per_w
        pltpu.sync_copy(idx_hbm.at[pl.ds(base * seq, k_per_w * seq)], idx_v)
        pltpu.sync_copy(pos_hbm, pos_v)

        def gathers(kchunk, slot):
            return [
                pltpu.make_async_copy(
                    tok_hbm.at[idx_v.at[pl.ds(kchunk * seq + off, width)]],
                    rows_v.at[slot, pl.ds(off, width)],
                    gsems[slot])
                for off, width in _SPLITS]

        def scatter(kchunk, slot):
            return pltpu.make_async_copy(
                rows_v.at[slot],
                out_hbm.at[base + kchunk, slice(None), pl.ds(0, hidden)],
                ssems[slot])

        def add_pos(slot):
            def add_row(i, _):
                for j in range(hidden // _LANES):
                    plsc.addupdate(
                        rows_v.at[slot, i, pl.ds(j * _LANES, _LANES)],
                        pos_v[pl.ds(i * hidden + j * _LANES, _LANES)])
                return 0
            lax.fori_loop(0, seq, add_row, 0, unroll=4)

        def visit(kchunk, b, tail=False, first=False):
            for c in gathers(kchunk, b):
                c.wait()
            add_pos(b)
            scatter(kchunk, b).start()
            if not first:
                scatter(kchunk - _SLAG, (b - _SLAG) % _NBUF).wait()
            if not tail:
                for c in gathers(kchunk + _LEAD, (b + _LEAD) % _NBUF):
                    c.start()

        for p in range(_LEAD):
            for c in gathers(p, p):
                c.start()
        for p in range(_LEAD):
            visit(p, p % _NBUF, first=p < _SLAG)

        def steady(kk, _):
            k0 = _LEAD + kk * _NBUF
            for off in range(_NBUF):
                visit(k0 + off, (_LEAD + off) % _NBUF)
            return 0
        lax.fori_loop(0, n_steady // _NBUF, steady, 0)

        for p in range(k_per_w - n_tail, k_per_w):
            visit(p, p % _NBUF, tail=p + _LEAD >= k_per_w)
        for p in range(k_per_w - _SLAG, k_per_w):
            scatter(p, p % _NBUF).wait()

    return k


def kernel(batch_input_idx, token_table, position_table):
    b, s = batch_input_idx.shape
    v, hidden = token_table.shape
    idx = batch_input_idx.astype(jnp.int32).reshape(-1)
    pos = position_table[:s].reshape(-1)
    tok = lax.optimization_barrier(token_table.reshape(-1)).reshape(v, hidden)
    out = _make_kernel(b, s, hidden)(idx, tok, pos)
    return out[:, :, :hidden]
